# baseline (speedup 1.0000x reference)
import functools

import jax
import jax.numpy as jnp
import numpy as np
from jax import lax
from jax.experimental import pallas as pl
from jax.experimental.pallas import tpu as pltpu
from jax.experimental.pallas import tpu_sc as plsc

N_FRAME = 50000
N_TFN = 10000
E_F2T = 50000
E_T2T = 320000
E_T2F = 50000
C_Z = 128
NUM_RBF = 16
C_S = 384
FEAT_DIM = 320

NW = 32
N_PAD = 10240
E_PAD = 50176
CA = 1536
CT = 848
CEMAX = 1664
EPS = 1e-08

_MU = [float(v) for v in np.linspace(0.0, 20.0, NUM_RBF)]
_INV_SIGMA = float(NUM_RBF) / 20.0
_S3 = float(np.sqrt(3.0))

_MESH = dict(core_axis_name="c", subcore_axis_name="s")
_PARAMS = pltpu.CompilerParams(needs_layout_passes=False)


def _rsqrt_fast(s):
    i = plsc.bitcast(s, jnp.int32)
    i = jnp.int32(0x5F3759DF) - lax.shift_right_arithmetic(i, 1)
    y = plsc.bitcast(i, jnp.float32)
    for _ in range(2):
        y = y * (1.5 - 0.5 * s * y * y)
    return y


def _edge_chunk(iA_h, iB_h, tabA, tabB, rbf_o, sh_o, base, ce, nv,
                idx_a, idx_b, rbf_buf, sh_buf):
    tAx, tAy, tAz = tabA
    tBx, tBy, tBz = tabB
    zeros16i = jnp.zeros((16,), jnp.int32)
    ones16 = jnp.ones((16,), jnp.float32)
    nan16 = jnp.full((16,), jnp.nan, jnp.float32)
    base = pl.multiple_of(base, 8)
    pltpu.sync_copy(iA_h.at[pl.ds(base, nv)], idx_a.at[pl.ds(0, nv)])
    pltpu.sync_copy(iB_h.at[pl.ds(base, nv)], idx_b.at[pl.ds(0, nv)])
    if nv < ce:
        def tz(i, _):
            slc = pl.ds(nv + i * 16, 16)
            idx_a[slc] = zeros16i
            idx_b[slc] = zeros16i
            return 0
        lax.fori_loop(0, (ce - nv) // 16, tz, 0)

    ng = ce // 16
    unr = 4 if ng % 4 == 0 else (2 if ng % 2 == 0 else 1)

    @plsc.parallel_loop(0, ng, 1, unroll=unr)
    def gbody(g):
        slc = pl.ds(g * 16, 16)
        ia = idx_a[slc]
        ib = idx_b[slc]
        ax = plsc.load_gather(tAx, [ia])
        ay = plsc.load_gather(tAy, [ia])
        az = plsc.load_gather(tAz, [ia])
        bx = plsc.load_gather(tBx, [ib])
        by = plsc.load_gather(tBy, [ib])
        bz = plsc.load_gather(tBz, [ib])
        vx = ax - bx
        vy = ay - by
        vz = az - bz
        s = vx * vx + vy * vy + vz * vz
        ex = vx + EPS
        ey = vy + EPS
        ez = vz + EPS
        se = ex * ex + ey * ey + ez * ez
        d = se * _rsqrt_fast(se)
        for k in range(NUM_RBF):
            t = (d - _MU[k]) * _INV_SIGMA
            rbf_buf[k, slc] = jnp.exp(-(t * t))
        inv = _rsqrt_fast(s)
        inv = jnp.where(s > 0.0, inv, nan16)
        sh_buf[0, slc] = ones16
        sh_buf[1, slc] = _S3 * vy * inv
        sh_buf[2, slc] = _S3 * vz * inv
        sh_buf[3, slc] = _S3 * vx * inv
    pltpu.sync_copy(rbf_buf.at[:, pl.ds(0, ce)], rbf_o.at[:, pl.ds(base, ce)])
    pltpu.sync_copy(sh_buf.at[:, pl.ds(0, ce)], sh_o.at[:, pl.ds(base, ce)])


def _short_phase(wid, iA_h, iB_h, tA, tB, rbf_o, sh_o, bufs):
    _edge_chunk(iA_h, iB_h, tA, tB, rbf_o, sh_o, wid * 1536, 1536, 1536,
                *bufs)

    @pl.when(wid < 6)
    def _():
        _edge_chunk(iA_h, iB_h, tA, tB, rbf_o, sh_o,
                    49152 + wid * 128, 128, 128, *bufs)

    @pl.when(wid == 6)
    def _():
        _edge_chunk(iA_h, iB_h, tA, tB, rbf_o, sh_o, 49920, 128, 80, *bufs)


def _rig8_body(rig_ref, out_ref):
    out_ref[0:3, :] = rig_ref[...].T


def _rig8(rigids):
    return pl.pallas_call(
        _rig8_body,
        grid=(pl.cdiv(E_PAD, 2048),),
        in_specs=[pl.BlockSpec((2048, 3), lambda i: (i, 0))],
        out_specs=pl.BlockSpec((8, 2048), lambda i: (0, i)),
        out_shape=jax.ShapeDtypeStruct((8, E_PAD), jnp.float32),
    )(rigids)


def _sc_a(rig8, f2t_i0, f2t_i1):
    mesh = plsc.VectorSubcoreMesh(**_MESH)
    out_type = (
        jax.ShapeDtypeStruct((8, N_PAD), jnp.float32),
        jax.ShapeDtypeStruct((NUM_RBF, E_PAD), jnp.float32),
        jax.ShapeDtypeStruct((8, E_PAD), jnp.float32),
        jax.ShapeDtypeStruct((N_PAD,), jnp.float32),
        jax.ShapeDtypeStruct((N_PAD,), jnp.float32),
        jax.ShapeDtypeStruct((N_PAD,), jnp.float32),
        jax.ShapeDtypeStruct((N_TFN,), jnp.float32),
        jax.ShapeDtypeStruct((N_TFN,), jnp.float32),
        jax.ShapeDtypeStruct((N_TFN,), jnp.float32),
    )
    scratch = [
        pltpu.VMEM_SHARED((N_PAD,), jnp.float32),
        pltpu.VMEM_SHARED((N_PAD,), jnp.float32),
        pltpu.VMEM_SHARED((N_PAD,), jnp.float32),
        pltpu.VMEM_SHARED((N_PAD,), jnp.float32),
        pltpu.VMEM((N_TFN,), jnp.float32),
        pltpu.VMEM((N_TFN,), jnp.float32),
        pltpu.VMEM((N_TFN,), jnp.float32),
        pltpu.VMEM((N_PAD,), jnp.float32),
        pltpu.VMEM((N_PAD,), jnp.float32),
        pltpu.VMEM((N_PAD,), jnp.float32),
        pltpu.VMEM((8, CA), jnp.float32),
        pltpu.VMEM((CA,), jnp.int32),
        pltpu.VMEM((CA,), jnp.float32),
        pltpu.VMEM((CA,), jnp.float32),
        pltpu.VMEM((CA,), jnp.float32),
        pltpu.VMEM((CA,), jnp.float32),
        pltpu.VMEM((CEMAX,), jnp.int32),
        pltpu.VMEM((CEMAX,), jnp.int32),
        pltpu.VMEM((NUM_RBF, CEMAX), jnp.float32),
        pltpu.VMEM((8, CEMAX), jnp.float32),
    ]

    @functools.partial(pl.kernel, out_type=out_type, mesh=mesh,
                       scratch_types=scratch, compiler_params=_PARAMS)
    def body(rig_h, f2t0_h, f2t1_h,
             tfn_o, f2t_rbf_o, f2t_sh_o, tbx_o, tby_o, tbz_o, trx_o, try_o,
             trz_o,
             acc_x, acc_y, acc_z, acc_c,
             tA_x, tA_y, tA_z, tB_x, tB_y, tB_z,
             rows2, scat_idx, col_x, col_y, col_z, col_c,
             idx_a, idx_b, rbf_buf, sh_buf):
        cid = lax.axis_index("c")
        sid = lax.axis_index("s")
        wid = sid * 2 + cid
        lane = lax.iota(jnp.int32, 16)
        zeros16 = jnp.zeros((16,), jnp.float32)
        zeros16i = jnp.zeros((16,), jnp.int32)
        ones16 = jnp.ones((16,), jnp.float32)

        def zb(i, _):
            col_c[pl.ds(i * 16, 16)] = zeros16
            return 0
        lax.fori_loop(0, 40, zb, 0)
        zslc = pl.ds(pl.multiple_of(sid * 640, 8), 640)
        for acc in (acc_x, acc_y, acc_z, acc_c):
            pltpu.sync_copy(col_c.at[pl.ds(0, 640)], acc.at[zslc])

        def ob(i, _):
            col_c[pl.ds(i * 16, 16)] = ones16
            return 0
        lax.fori_loop(0, CA // 16, ob, 0)
        plsc.subcore_barrier()

        def scat_chunk(base, nv, nc):
            base = pl.multiple_of(base, 8)
            pltpu.sync_copy(f2t0_h.at[pl.ds(base, nv)],
                            scat_idx.at[pl.ds(0, nv)])
            pltpu.sync_copy(rig_h.at[:, pl.ds(base, nc)],
                            rows2.at[:, pl.ds(0, nc)])

            ng = nv // 16

            @plsc.parallel_loop(0, ng, 1, unroll=(2 if ng % 2 == 0 else 1))
            def cb(g):
                slc = pl.ds(g * 16, 16)
                col_x[slc] = rows2[0, slc]
                col_y[slc] = rows2[1, slc]
                col_z[slc] = rows2[2, slc]
            if nv < CA:
                def tz(i, _):
                    slc = pl.ds(nv + i * 16, 16)
                    scat_idx[slc] = zeros16i
                    col_x[slc] = zeros16
                    col_y[slc] = zeros16
                    col_z[slc] = zeros16
                    col_c[slc] = zeros16
                    return 0
                lax.fori_loop(0, (CA - nv) // 16, tz, 0)
            pltpu.sync_copy(col_x, acc_x.at[scat_idx], add=True)
            pltpu.sync_copy(col_y, acc_y.at[scat_idx], add=True)
            pltpu.sync_copy(col_z, acc_z.at[scat_idx], add=True)
            pltpu.sync_copy(col_c, acc_c.at[scat_idx], add=True)

        scat_chunk(sid * CA, CA, CA)
        scat_chunk((sid + 16) * CA, CA, CA)

        @pl.when(sid == 15)
        def _():
            scat_chunk(32 * CA, CT, 896)

        plsc.subcore_barrier()

        pltpu.sync_copy(acc_x, tB_x)
        pltpu.sync_copy(acc_y, tB_y)
        pltpu.sync_copy(acc_z, tB_z)

        for q in range(8):
            pltpu.sync_copy(acc_c.at[pl.ds(q * 1280, 1280)],
                            col_c.at[pl.ds(0, 1280)])

            def dbody(i, _, q=q):
                slc = pl.ds(i * 16, 16)
                gslc = pl.ds(q * 1280 + i * 16, 16)
                invc = 1.0 / jnp.maximum(col_c[slc], 1.0)
                tB_x[gslc] = tB_x[gslc] * invc
                tB_y[gslc] = tB_y[gslc] * invc
                tB_z[gslc] = tB_z[gslc] * invc
                return 0
            lax.fori_loop(0, 80, dbody, 0)

        @pl.when(cid == 0)
        def _():
            cbase = pl.multiple_of(sid * 640, 8)

            def tb(j, _):
                slc = pl.ds(cbase + j * 16, 16)
                dst = pl.ds(j * 16, 16)
                sh_buf[0, dst] = tB_x[slc]
                sh_buf[1, dst] = tB_y[slc]
                sh_buf[2, dst] = tB_z[slc]
                return 0
            lax.fori_loop(0, 40, tb, 0)
            pltpu.sync_copy(sh_buf.at[:, pl.ds(0, 640)],
                            tfn_o.at[:, pl.ds(cbase, 640)])

        for tc in range(7):
            trows = CA if tc < 6 else N_TFN - 6 * CA
            ncols = CA if tc < 6 else 896
            tbase = tc * CA
            pltpu.sync_copy(rig_h.at[:, pl.ds(tbase, ncols)],
                            rows2.at[:, pl.ds(0, ncols)])

            tng = trows // 16

            @plsc.parallel_loop(0, tng, 1,
                                unroll=(2 if tng % 2 == 0 else 1))
            def tcb(g, tbase=tbase):
                slc = pl.ds(g * 16, 16)
                dslc = pl.ds(tbase + g * 16, 16)
                tA_x[dslc] = rows2[0, slc]
                tA_y[dslc] = rows2[1, slc]
                tA_z[dslc] = rows2[2, slc]

        @pl.when(wid == 1)
        def _():
            pltpu.sync_copy(tB_x, tbx_o)
            pltpu.sync_copy(tB_y, tby_o)
            pltpu.sync_copy(tB_z, tbz_o)

        @pl.when(wid == 3)
        def _():
            pltpu.sync_copy(tA_x, trx_o)
            pltpu.sync_copy(tA_y, try_o)
            pltpu.sync_copy(tA_z, trz_o)

        bufs = (idx_a, idx_b, rbf_buf, sh_buf)
        _short_phase(wid, f2t0_h, f2t1_h, (tA_x, tA_y, tA_z),
                     (tB_x, tB_y, tB_z), f2t_rbf_o, f2t_sh_o, bufs)

    return body(rig8, f2t_i0, f2t_i1)


def _sc_b(tbx, tby, tbz, trx, try_, trz, t2t_i0, t2t_i1, t2f_i0, t2f_i1):
    mesh = plsc.VectorSubcoreMesh(**_MESH)
    out_type = (
        jax.ShapeDtypeStruct((NUM_RBF, E_T2T), jnp.float32),
        jax.ShapeDtypeStruct((8, E_T2T), jnp.float32),
        jax.ShapeDtypeStruct((NUM_RBF, E_PAD), jnp.float32),
        jax.ShapeDtypeStruct((8, E_PAD), jnp.float32),
    )
    scratch = [
        pltpu.VMEM((N_TFN,), jnp.float32),
        pltpu.VMEM((N_TFN,), jnp.float32),
        pltpu.VMEM((N_TFN,), jnp.float32),
        pltpu.VMEM((N_PAD,), jnp.float32),
        pltpu.VMEM((N_PAD,), jnp.float32),
        pltpu.VMEM((N_PAD,), jnp.float32),
        pltpu.VMEM((CEMAX,), jnp.int32),
        pltpu.VMEM((CEMAX,), jnp.int32),
        pltpu.VMEM((NUM_RBF, CEMAX), jnp.float32),
        pltpu.VMEM((8, CEMAX), jnp.float32),
    ]

    @functools.partial(pl.kernel, out_type=out_type, mesh=mesh,
                       scratch_types=scratch, compiler_params=_PARAMS)
    def body(tbx_h, tby_h, tbz_h, trx_h, try_h, trz_h,
             t2t0_h, t2t1_h, t2f0_h, t2f1_h,
             t2t_rbf_o, t2t_sh_o, t2f_rbf_o, t2f_sh_o,
             tA_x, tA_y, tA_z, tB_x, tB_y, tB_z,
             idx_a, idx_b, rbf_buf, sh_buf):
        cid = lax.axis_index("c")
        sid = lax.axis_index("s")
        wid = sid * 2 + cid

        pltpu.sync_copy(tbx_h, tB_x)
        pltpu.sync_copy(tby_h, tB_y)
        pltpu.sync_copy(tbz_h, tB_z)
        pltpu.sync_copy(trx_h, tA_x)
        pltpu.sync_copy(try_h, tA_y)
        pltpu.sync_copy(trz_h, tA_z)

        tabA = (tA_x, tA_y, tA_z)
        tabB = (tB_x, tB_y, tB_z)
        bufs = (idx_a, idx_b, rbf_buf, sh_buf)

        def t2t_chunk(c, _):
            _edge_chunk(t2t0_h, t2t1_h, tabB, tabB, t2t_rbf_o, t2t_sh_o,
                        wid * 9984 + c * 1664, 1664, 1664, *bufs)
            return 0
        lax.fori_loop(0, 6, t2t_chunk, 0)

        @pl.when(wid < 4)
        def _():
            _edge_chunk(t2t0_h, t2t1_h, tabB, tabB, t2t_rbf_o, t2t_sh_o,
                        319488 + wid * 128, 128, 128, *bufs)

        _short_phase(wid, t2f0_h, t2f1_h, tabB, tabA, t2f_rbf_o, t2f_sh_o,
                     bufs)

    return body(tbx, tby, tbz, trx, try_, trz, t2t_i0, t2t_i1, t2f_i0,
                t2f_i1)


_BLK = 2048


def _mlp_body(ef_ref, rbft_ref, w1a_ref, w1b_ref, b1_ref, w2_ref, b2_ref,
              g_ref, be_ref, out_ref):
    h = jnp.dot(ef_ref[...], w1a_ref[...],
                preferred_element_type=jnp.float32)
    h = h + lax.dot_general(rbft_ref[...], w1b_ref[...],
                            (((0,), (0,)), ((), ())),
                            preferred_element_type=jnp.float32)
    h = jnp.maximum(h + b1_ref[...], 0.0)
    o = jnp.dot(h, w2_ref[...],
                preferred_element_type=jnp.float32) + b2_ref[...]
    mu = jnp.mean(o, axis=-1, keepdims=True)
    c = o - mu
    var = jnp.mean(c * c, axis=-1, keepdims=True)
    out_ref[...] = c * lax.rsqrt(var + 1e-05) * g_ref[...] + be_ref[...]


def _mlp(ef, rbft, w1a, w1b, b1, w2, b2, g, be):
    grid = pl.cdiv(E_F2T, _BLK)
    return pl.pallas_call(
        _mlp_body,
        grid=(grid,),
        in_specs=[
            pl.BlockSpec((_BLK, C_Z), lambda i: (i, 0)),
            pl.BlockSpec((NUM_RBF, _BLK), lambda i: (0, i)),
            pl.BlockSpec((C_Z, 2 * C_Z), lambda i: (0, 0)),
            pl.BlockSpec((NUM_RBF, 2 * C_Z), lambda i: (0, 0)),
            pl.BlockSpec((1, 2 * C_Z), lambda i: (0, 0)),
            pl.BlockSpec((2 * C_Z, C_Z), lambda i: (0, 0)),
            pl.BlockSpec((1, C_Z), lambda i: (0, 0)),
            pl.BlockSpec((1, C_Z), lambda i: (0, 0)),
            pl.BlockSpec((1, C_Z), lambda i: (0, 0)),
        ],
        out_specs=pl.BlockSpec((_BLK, C_Z), lambda i: (i, 0)),
        out_shape=jax.ShapeDtypeStruct((E_F2T, C_Z), jnp.float32),
    )(ef, rbft, w1a, w1b, b1, w2, b2, g, be)


def kernel(frame_features, tfn_features, frame2tfn_edge_features,
           tfn2tfn_edge_features, tfn2frame_edge_features, rigids,
           frame2tfn_edge_index, tfn2tfn_edge_index, tfn2frame_edge_index,
           res_mask, W1, b1, W2, b2, ln_g, ln_b):
    (tfn8, f2t_rbf_t, f2t_sh8, tbx, tby, tbz, trx, try_, trz) = _sc_a(
        _rig8(rigids),
        frame2tfn_edge_index[0].astype(jnp.int32),
        frame2tfn_edge_index[1].astype(jnp.int32))

    (t2t_rbf_t, t2t_sh8, t2f_rbf_t, t2f_sh8) = _sc_b(
        tbx, tby, tbz, trx, try_, trz,
        tfn2tfn_edge_index[0].astype(jnp.int32),
        tfn2tfn_edge_index[1].astype(jnp.int32),
        tfn2frame_edge_index[0].astype(jnp.int32),
        tfn2frame_edge_index[1].astype(jnp.int32))

    f2t_updated = _mlp(frame2tfn_edge_features, f2t_rbf_t,
                       W1[:C_Z], W1[C_Z:], b1.reshape(1, -1),
                       W2, b2.reshape(1, -1), ln_g.reshape(1, -1),
                       ln_b.reshape(1, -1))

    return (f2t_updated,
            f2t_sh8[:4, :E_F2T].T,
            t2t_rbf_t.T,
            t2t_sh8[:4].T,
            t2f_rbf_t[:, :E_T2F].T,
            t2f_sh8[:4, :E_T2F].T,
            tfn8[:3, :N_TFN].T)


# speedup vs baseline: 1.0948x; 1.0948x over previous
import functools

import jax
import jax.numpy as jnp
import numpy as np
from jax import lax
from jax.experimental import pallas as pl
from jax.experimental.pallas import tpu as pltpu
from jax.experimental.pallas import tpu_sc as plsc

N_FRAME = 50000
N_TFN = 10000
E_F2T = 50000
E_T2T = 320000
E_T2F = 50000
C_Z = 128
NUM_RBF = 16
C_S = 384
FEAT_DIM = 320

NW = 32
N_PAD = 10240
E_PAD = 50176
CA = 1568
CT = 1392
CEMAX = 1664
EPS = 1e-08

_MU = [float(v) for v in np.linspace(0.0, 20.0, NUM_RBF)]
_INV_SIGMA = float(NUM_RBF) / 20.0
_S3 = float(np.sqrt(3.0))

_MESH = dict(core_axis_name="c", subcore_axis_name="s")
_PARAMS = pltpu.CompilerParams(needs_layout_passes=False)


def _rsqrt_fast(s):
    i = plsc.bitcast(s, jnp.int32)
    i = jnp.int32(0x5F3759DF) - lax.shift_right_arithmetic(i, 1)
    y = plsc.bitcast(i, jnp.float32)
    for _ in range(2):
        y = y * (1.5 - 0.5 * s * y * y)
    return y


def _edge_chunk(iA_h, iB_h, tabA, tabB, rbf_o, sh_o, base, ce, nv,
                idx_a, idx_b, rbf_buf, sh_buf):
    tAx, tAy, tAz = tabA
    tBx, tBy, tBz = tabB
    zeros16i = jnp.zeros((16,), jnp.int32)
    ones16 = jnp.ones((16,), jnp.float32)
    nan16 = jnp.full((16,), jnp.nan, jnp.float32)
    base = pl.multiple_of(base, 8)
    pltpu.sync_copy(iA_h.at[pl.ds(base, nv)], idx_a.at[pl.ds(0, nv)])
    pltpu.sync_copy(iB_h.at[pl.ds(base, nv)], idx_b.at[pl.ds(0, nv)])
    if nv < ce:
        def tz(i, _):
            slc = pl.ds(nv + i * 16, 16)
            idx_a[slc] = zeros16i
            idx_b[slc] = zeros16i
            return 0
        lax.fori_loop(0, (ce - nv) // 16, tz, 0)

    ng = ce // 16
    unr = 4 if ng % 4 == 0 else (2 if ng % 2 == 0 else 1)

    @plsc.parallel_loop(0, ng, 1, unroll=unr)
    def gbody(g):
        slc = pl.ds(g * 16, 16)
        ia = idx_a[slc]
        ib = idx_b[slc]
        ax = plsc.load_gather(tAx, [ia])
        ay = plsc.load_gather(tAy, [ia])
        az = plsc.load_gather(tAz, [ia])
        bx = plsc.load_gather(tBx, [ib])
        by = plsc.load_gather(tBy, [ib])
        bz = plsc.load_gather(tBz, [ib])
        vx = ax - bx
        vy = ay - by
        vz = az - bz
        s = vx * vx + vy * vy + vz * vz
        ex = vx + EPS
        ey = vy + EPS
        ez = vz + EPS
        se = ex * ex + ey * ey + ez * ez
        d = se * _rsqrt_fast(se)
        for k in range(NUM_RBF):
            t = (d - _MU[k]) * _INV_SIGMA
            rbf_buf[k, slc] = jnp.exp(-(t * t))
        inv = _rsqrt_fast(s)
        inv = jnp.where(s > 0.0, inv, nan16)
        sh_buf[0, slc] = ones16
        sh_buf[1, slc] = _S3 * vy * inv
        sh_buf[2, slc] = _S3 * vz * inv
        sh_buf[3, slc] = _S3 * vx * inv
    pltpu.sync_copy(rbf_buf.at[:, pl.ds(0, ce)], rbf_o.at[:, pl.ds(base, ce)])
    pltpu.sync_copy(sh_buf.at[:, pl.ds(0, ce)], sh_o.at[:, pl.ds(base, ce)])


def _short_phase(wid, iA_h, iB_h, tA, tB, rbf_o, sh_o, bufs):
    _edge_chunk(iA_h, iB_h, tA, tB, rbf_o, sh_o, wid * 1536, 1536, 1536,
                *bufs)

    @pl.when(wid < 6)
    def _():
        _edge_chunk(iA_h, iB_h, tA, tB, rbf_o, sh_o,
                    49152 + wid * 128, 128, 128, *bufs)

    @pl.when(wid == 6)
    def _():
        _edge_chunk(iA_h, iB_h, tA, tB, rbf_o, sh_o, 49920, 128, 80, *bufs)


def _sc_a(rig_flat, f2t_i0, f2t_i1):
    mesh = plsc.VectorSubcoreMesh(**_MESH)
    out_type = (
        jax.ShapeDtypeStruct((8, N_PAD), jnp.float32),
        jax.ShapeDtypeStruct((NUM_RBF, E_PAD), jnp.float32),
        jax.ShapeDtypeStruct((8, E_PAD), jnp.float32),
        jax.ShapeDtypeStruct((N_PAD,), jnp.float32),
        jax.ShapeDtypeStruct((N_PAD,), jnp.float32),
        jax.ShapeDtypeStruct((N_PAD,), jnp.float32),
        jax.ShapeDtypeStruct((N_TFN,), jnp.float32),
        jax.ShapeDtypeStruct((N_TFN,), jnp.float32),
        jax.ShapeDtypeStruct((N_TFN,), jnp.float32),
    )
    scratch = [
        pltpu.VMEM_SHARED((N_PAD,), jnp.float32),
        pltpu.VMEM_SHARED((N_PAD,), jnp.float32),
        pltpu.VMEM_SHARED((N_PAD,), jnp.float32),
        pltpu.VMEM_SHARED((N_PAD,), jnp.float32),
        pltpu.VMEM((N_TFN,), jnp.float32),
        pltpu.VMEM((N_TFN,), jnp.float32),
        pltpu.VMEM((N_TFN,), jnp.float32),
        pltpu.VMEM((N_PAD,), jnp.float32),
        pltpu.VMEM((N_PAD,), jnp.float32),
        pltpu.VMEM((N_PAD,), jnp.float32),
        pltpu.VMEM((N_PAD,), jnp.float32),
        pltpu.VMEM((CA * 3,), jnp.float32),
        pltpu.VMEM((CA,), jnp.int32),
        pltpu.VMEM((CA,), jnp.float32),
        pltpu.VMEM((CA,), jnp.float32),
        pltpu.VMEM((CA,), jnp.float32),
        pltpu.VMEM((CA,), jnp.float32),
        pltpu.VMEM((CEMAX,), jnp.int32),
        pltpu.VMEM((CEMAX,), jnp.int32),
        pltpu.VMEM((NUM_RBF, CEMAX), jnp.float32),
        pltpu.VMEM((8, CEMAX), jnp.float32),
    ]

    @functools.partial(pl.kernel, out_type=out_type, mesh=mesh,
                       scratch_types=scratch, compiler_params=_PARAMS)
    def body(rig_h, f2t0_h, f2t1_h,
             tfn_o, f2t_rbf_o, f2t_sh_o, tbx_o, tby_o, tbz_o, trx_o, try_o,
             trz_o,
             acc_x, acc_y, acc_z, acc_c,
             tA_x, tA_y, tA_z, tB_x, tB_y, tB_z, tB_c,
             rows_buf, scat_idx, col_x, col_y, col_z, col_c,
             idx_a, idx_b, rbf_buf, sh_buf):
        cid = lax.axis_index("c")
        sid = lax.axis_index("s")
        wid = sid * 2 + cid
        lane = lax.iota(jnp.int32, 16)
        zeros16 = jnp.zeros((16,), jnp.float32)
        zeros16i = jnp.zeros((16,), jnp.int32)
        ones16 = jnp.ones((16,), jnp.float32)

        def zb(i, _):
            col_c[pl.ds(i * 16, 16)] = zeros16
            return 0
        lax.fori_loop(0, 40, zb, 0)
        zslc = pl.ds(pl.multiple_of(sid * 640, 8), 640)
        for acc in (acc_x, acc_y, acc_z, acc_c):
            pltpu.sync_copy(col_c.at[pl.ds(0, 640)], acc.at[zslc])

        def ob(i, _):
            col_c[pl.ds(i * 16, 16)] = ones16
            return 0
        lax.fori_loop(0, CA // 16, ob, 0)
        plsc.subcore_barrier()

        def scat_chunk(base, nv):
            base = pl.multiple_of(base, 8)
            pltpu.sync_copy(f2t0_h.at[pl.ds(base, nv)],
                            scat_idx.at[pl.ds(0, nv)])
            pltpu.sync_copy(rig_h.at[pl.ds(base * 3, nv * 3)],
                            rows_buf.at[pl.ds(0, nv * 3)])

            ng = nv // 16

            @plsc.parallel_loop(0, ng, 1, unroll=(2 if ng % 2 == 0 else 1))
            def cb(g):
                e3 = 3 * (g * 16 + lane)
                slc = pl.ds(g * 16, 16)
                col_x[slc] = plsc.load_gather(rows_buf, [e3])
                col_y[slc] = plsc.load_gather(rows_buf, [e3 + 1])
                col_z[slc] = plsc.load_gather(rows_buf, [e3 + 2])
            if nv < CA:
                def tz(i, _):
                    slc = pl.ds(nv + i * 16, 16)
                    scat_idx[slc] = zeros16i
                    col_x[slc] = zeros16
                    col_y[slc] = zeros16
                    col_z[slc] = zeros16
                    col_c[slc] = zeros16
                    return 0
                lax.fori_loop(0, (CA - nv) // 16, tz, 0)
            pltpu.sync_copy(col_x, acc_x.at[scat_idx], add=True)
            pltpu.sync_copy(col_y, acc_y.at[scat_idx], add=True)
            pltpu.sync_copy(col_z, acc_z.at[scat_idx], add=True)
            pltpu.sync_copy(col_c, acc_c.at[scat_idx], add=True)

        scat_chunk(sid * CA, CA)

        @pl.when(sid < 15)
        def _():
            scat_chunk((sid + 16) * CA, CA)

        @pl.when(sid == 15)
        def _():
            scat_chunk(31 * CA, CT)

        plsc.subcore_barrier()

        pltpu.sync_copy(acc_x, tB_x)
        pltpu.sync_copy(acc_y, tB_y)
        pltpu.sync_copy(acc_z, tB_z)
        pltpu.sync_copy(acc_c, tB_c)

        def dbody(i, _):
            slc = pl.ds(i * 16, 16)
            invc = 1.0 / jnp.maximum(tB_c[slc], 1.0)
            tB_x[slc] = tB_x[slc] * invc
            tB_y[slc] = tB_y[slc] * invc
            tB_z[slc] = tB_z[slc] * invc
            return 0
        lax.fori_loop(0, N_PAD // 16, dbody, 0)

        @pl.when(cid == 0)
        def _():
            cbase = pl.multiple_of(sid * 640, 8)

            def tb(j, _):
                slc = pl.ds(cbase + j * 16, 16)
                dst = pl.ds(j * 16, 16)
                sh_buf[0, dst] = tB_x[slc]
                sh_buf[1, dst] = tB_y[slc]
                sh_buf[2, dst] = tB_z[slc]
                return 0
            lax.fori_loop(0, 40, tb, 0)
            pltpu.sync_copy(sh_buf.at[:, pl.ds(0, 640)],
                            tfn_o.at[:, pl.ds(cbase, 640)])

        for tc in range(7):
            trows = CA if tc < 6 else N_TFN - 6 * CA
            tbase = tc * CA
            pltpu.sync_copy(rig_h.at[pl.ds(tbase * 3, trows * 3)],
                            rows_buf.at[pl.ds(0, trows * 3)])

            tng = trows // 16

            @plsc.parallel_loop(0, tng, 1,
                                unroll=(2 if tng % 2 == 0 else 1))
            def tcb(g, tbase=tbase):
                e3 = 3 * (g * 16 + lane)
                slc = pl.ds(tbase + g * 16, 16)
                tA_x[slc] = plsc.load_gather(rows_buf, [e3])
                tA_y[slc] = plsc.load_gather(rows_buf, [e3 + 1])
                tA_z[slc] = plsc.load_gather(rows_buf, [e3 + 2])

        @pl.when(wid == 1)
        def _():
            pltpu.sync_copy(tB_x, tbx_o)
            pltpu.sync_copy(tB_y, tby_o)
            pltpu.sync_copy(tB_z, tbz_o)

        @pl.when(wid == 3)
        def _():
            pltpu.sync_copy(tA_x, trx_o)
            pltpu.sync_copy(tA_y, try_o)
            pltpu.sync_copy(tA_z, trz_o)

        bufs = (idx_a, idx_b, rbf_buf, sh_buf)
        _short_phase(wid, f2t0_h, f2t1_h, (tA_x, tA_y, tA_z),
                     (tB_x, tB_y, tB_z), f2t_rbf_o, f2t_sh_o, bufs)

    return body(rig_flat, f2t_i0, f2t_i1)


def _sc_b(tbx, tby, tbz, trx, try_, trz, t2t_i0, t2t_i1, t2f_i0, t2f_i1):
    mesh = plsc.VectorSubcoreMesh(**_MESH)
    out_type = (
        jax.ShapeDtypeStruct((NUM_RBF, E_T2T), jnp.float32),
        jax.ShapeDtypeStruct((8, E_T2T), jnp.float32),
        jax.ShapeDtypeStruct((NUM_RBF, E_PAD), jnp.float32),
        jax.ShapeDtypeStruct((8, E_PAD), jnp.float32),
    )
    scratch = [
        pltpu.VMEM((N_TFN,), jnp.float32),
        pltpu.VMEM((N_TFN,), jnp.float32),
        pltpu.VMEM((N_TFN,), jnp.float32),
        pltpu.VMEM((N_PAD,), jnp.float32),
        pltpu.VMEM((N_PAD,), jnp.float32),
        pltpu.VMEM((N_PAD,), jnp.float32),
        pltpu.VMEM((CEMAX,), jnp.int32),
        pltpu.VMEM((CEMAX,), jnp.int32),
        pltpu.VMEM((NUM_RBF, CEMAX), jnp.float32),
        pltpu.VMEM((8, CEMAX), jnp.float32),
    ]

    @functools.partial(pl.kernel, out_type=out_type, mesh=mesh,
                       scratch_types=scratch, compiler_params=_PARAMS)
    def body(tbx_h, tby_h, tbz_h, trx_h, try_h, trz_h,
             t2t0_h, t2t1_h, t2f0_h, t2f1_h,
             t2t_rbf_o, t2t_sh_o, t2f_rbf_o, t2f_sh_o,
             tA_x, tA_y, tA_z, tB_x, tB_y, tB_z,
             idx_a, idx_b, rbf_buf, sh_buf):
        cid = lax.axis_index("c")
        sid = lax.axis_index("s")
        wid = sid * 2 + cid

        pltpu.sync_copy(tbx_h, tB_x)
        pltpu.sync_copy(tby_h, tB_y)
        pltpu.sync_copy(tbz_h, tB_z)
        pltpu.sync_copy(trx_h, tA_x)
        pltpu.sync_copy(try_h, tA_y)
        pltpu.sync_copy(trz_h, tA_z)

        tabA = (tA_x, tA_y, tA_z)
        tabB = (tB_x, tB_y, tB_z)
        bufs = (idx_a, idx_b, rbf_buf, sh_buf)

        def t2t_chunk(c, _):
            _edge_chunk(t2t0_h, t2t1_h, tabB, tabB, t2t_rbf_o, t2t_sh_o,
                        wid * 9984 + c * 1664, 1664, 1664, *bufs)
            return 0
        lax.fori_loop(0, 6, t2t_chunk, 0)

        @pl.when(wid < 4)
        def _():
            _edge_chunk(t2t0_h, t2t1_h, tabB, tabB, t2t_rbf_o, t2t_sh_o,
                        319488 + wid * 128, 128, 128, *bufs)

        _short_phase(wid, t2f0_h, t2f1_h, tabB, tabA, t2f_rbf_o, t2f_sh_o,
                     bufs)

    return body(tbx, tby, tbz, trx, try_, trz, t2t_i0, t2t_i1, t2f_i0,
                t2f_i1)


_BLK = 2048


def _mlp_body(ef_ref, rbft_ref, w1a_ref, w1b_ref, b1_ref, w2_ref, b2_ref,
              g_ref, be_ref, out_ref):
    h = jnp.dot(ef_ref[...], w1a_ref[...],
                preferred_element_type=jnp.float32)
    h = h + lax.dot_general(rbft_ref[...], w1b_ref[...],
                            (((0,), (0,)), ((), ())),
                            preferred_element_type=jnp.float32)
    h = jnp.maximum(h + b1_ref[...], 0.0)
    o = jnp.dot(h, w2_ref[...],
                preferred_element_type=jnp.float32) + b2_ref[...]
    mu = jnp.mean(o, axis=-1, keepdims=True)
    c = o - mu
    var = jnp.mean(c * c, axis=-1, keepdims=True)
    out_ref[...] = c * lax.rsqrt(var + 1e-05) * g_ref[...] + be_ref[...]


def _mlp(ef, rbft, w1a, w1b, b1, w2, b2, g, be):
    grid = pl.cdiv(E_F2T, _BLK)
    return pl.pallas_call(
        _mlp_body,
        grid=(grid,),
        in_specs=[
            pl.BlockSpec((_BLK, C_Z), lambda i: (i, 0)),
            pl.BlockSpec((NUM_RBF, _BLK), lambda i: (0, i)),
            pl.BlockSpec((C_Z, 2 * C_Z), lambda i: (0, 0)),
            pl.BlockSpec((NUM_RBF, 2 * C_Z), lambda i: (0, 0)),
            pl.BlockSpec((1, 2 * C_Z), lambda i: (0, 0)),
            pl.BlockSpec((2 * C_Z, C_Z), lambda i: (0, 0)),
            pl.BlockSpec((1, C_Z), lambda i: (0, 0)),
            pl.BlockSpec((1, C_Z), lambda i: (0, 0)),
            pl.BlockSpec((1, C_Z), lambda i: (0, 0)),
        ],
        out_specs=pl.BlockSpec((_BLK, C_Z), lambda i: (i, 0)),
        out_shape=jax.ShapeDtypeStruct((E_F2T, C_Z), jnp.float32),
    )(ef, rbft, w1a, w1b, b1, w2, b2, g, be)


def kernel(frame_features, tfn_features, frame2tfn_edge_features,
           tfn2tfn_edge_features, tfn2frame_edge_features, rigids,
           frame2tfn_edge_index, tfn2tfn_edge_index, tfn2frame_edge_index,
           res_mask, W1, b1, W2, b2, ln_g, ln_b):
    (tfn8, f2t_rbf_t, f2t_sh8, tbx, tby, tbz, trx, try_, trz) = _sc_a(
        rigids.reshape(N_FRAME * 3),
        frame2tfn_edge_index[0].astype(jnp.int32),
        frame2tfn_edge_index[1].astype(jnp.int32))

    (t2t_rbf_t, t2t_sh8, t2f_rbf_t, t2f_sh8) = _sc_b(
        tbx, tby, tbz, trx, try_, trz,
        tfn2tfn_edge_index[0].astype(jnp.int32),
        tfn2tfn_edge_index[1].astype(jnp.int32),
        tfn2frame_edge_index[0].astype(jnp.int32),
        tfn2frame_edge_index[1].astype(jnp.int32))

    f2t_updated = _mlp(frame2tfn_edge_features, f2t_rbf_t,
                       W1[:C_Z], W1[C_Z:], b1.reshape(1, -1),
                       W2, b2.reshape(1, -1), ln_g.reshape(1, -1),
                       ln_b.reshape(1, -1))

    return (f2t_updated,
            f2t_sh8[:4, :E_F2T].T,
            t2t_rbf_t.T,
            t2t_sh8[:4].T,
            t2f_rbf_t[:, :E_T2F].T,
            t2f_sh8[:4, :E_T2F].T,
            tfn8[:3, :N_TFN].T)
